# baseline (speedup 1.0000x reference)
import functools

import jax
import jax.numpy as jnp
from jax import lax
from jax.experimental import pallas as pl
from jax.experimental.pallas import tpu as pltpu
from jax.experimental.pallas import tpu_sc as plsc

BATCH = 16384
EMBED = 128

_NC, _NS = 2, 16
_NW = _NC * _NS

_NSPLIT = 2
_HALF = BATCH // _NSPLIT
_B_PER_W = _HALF // _NW


def _make_gather(split):
    mesh = plsc.VectorSubcoreMesh(core_axis_name="c", subcore_axis_name="s")

    @functools.partial(
        pl.kernel,
        mesh=mesh,
        out_type=[
            jax.ShapeDtypeStruct((_HALF, EMBED), jnp.float32),
            jax.ShapeDtypeStruct((_HALF, EMBED), jnp.float32),
        ],
        scratch_types=[
            pltpu.VMEM((_B_PER_W,), jnp.int32),
            pltpu.VMEM((_B_PER_W, EMBED), jnp.float32),
            pltpu.SemaphoreType.DMA,
        ],
    )
    def gather_k(users_hbm, movies_hbm, ut_hbm, mt_hbm, u_out, m_out,
                 idx_v, rows_v, sem):
        wid = lax.axis_index("s") * _NC + lax.axis_index("c")
        out_base = wid * _B_PER_W
        in_base = split * _HALF + out_base
        pltpu.sync_copy(users_hbm.at[pl.ds(in_base, _B_PER_W)], idx_v)
        pltpu.async_copy(ut_hbm.at[idx_v], rows_v, sem).wait()
        pltpu.sync_copy(rows_v, u_out.at[pl.ds(out_base, _B_PER_W)])
        pltpu.sync_copy(movies_hbm.at[pl.ds(in_base, _B_PER_W)], idx_v)
        pltpu.async_copy(mt_hbm.at[idx_v], rows_v, sem).wait()
        pltpu.sync_copy(rows_v, m_out.at[pl.ds(out_base, _B_PER_W)])

    return gather_k


_gathers = [_make_gather(s) for s in range(_NSPLIT)]

_BLK = 2048


def _mlp_body(u_ref, m_ref, w1a_ref, w1b_ref, b1_ref, w2_ref, b2_ref,
              w3t_ref, b3_ref, o_ref):
    h = jnp.dot(u_ref[...], w1a_ref[...], preferred_element_type=jnp.float32)
    h += jnp.dot(m_ref[...], w1b_ref[...], preferred_element_type=jnp.float32)
    h = jnp.maximum(h + b1_ref[...], 0.0)
    h = jnp.maximum(
        jnp.dot(h, w2_ref[...], preferred_element_type=jnp.float32)
        + b2_ref[...], 0.0)
    o_ref[...] = lax.dot_general(
        w3t_ref[...], h, (((1,), (1,)), ((), ())),
        preferred_element_type=jnp.float32) + b3_ref[0, 0]


def _mlp(u, m, w1a, w1b, b1r, W2, b2r, w3t, b3r):
    grid = _HALF // _BLK
    return pl.pallas_call(
        _mlp_body,
        grid=(grid,),
        in_specs=[
            pl.BlockSpec((_BLK, EMBED), lambda i: (i, 0)),
            pl.BlockSpec((_BLK, EMBED), lambda i: (i, 0)),
            pl.BlockSpec((EMBED, 128), lambda i: (0, 0)),
            pl.BlockSpec((EMBED, 128), lambda i: (0, 0)),
            pl.BlockSpec((1, 128), lambda i: (0, 0)),
            pl.BlockSpec((128, 64), lambda i: (0, 0)),
            pl.BlockSpec((1, 64), lambda i: (0, 0)),
            pl.BlockSpec((1, 64), lambda i: (0, 0)),
            pl.BlockSpec((1, 1), lambda i: (0, 0)),
        ],
        out_specs=pl.BlockSpec((1, _BLK), lambda i: (0, i)),
        out_shape=jax.ShapeDtypeStruct((1, _HALF), jnp.float32),
    )(u, m, w1a, w1b, b1r, W2, b2r, w3t, b3r)


def kernel(users, movies, user_table, movie_table, W1, b1, W2, b2, W3, b3):
    users = users.astype(jnp.int32)
    movies = movies.astype(jnp.int32)
    w1a, w1b = W1[:EMBED], W1[EMBED:]
    b1r, b2r, b3r = b1.reshape(1, 128), b2.reshape(1, 64), b3.reshape(1, 1)
    w3t = W3.reshape(1, 64)
    gathered = [g(users, movies, user_table, movie_table) for g in _gathers]
    rows = [_mlp(u, m, w1a, w1b, b1r, W2, b2r, w3t, b3r)
            for u, m in gathered]
    return jnp.concatenate(rows, axis=1).reshape(BATCH, 1)


# speedup vs baseline: 1.0845x; 1.0845x over previous
import functools

import jax
import jax.numpy as jnp
from jax import lax
from jax.experimental import pallas as pl
from jax.experimental.pallas import tpu as pltpu
from jax.experimental.pallas import tpu_sc as plsc

BATCH = 16384
EMBED = 128

_NC, _NS = 2, 16
_NW = _NC * _NS
_B_PER_W = BATCH // _NW


def _make_gather():
    mesh = plsc.VectorSubcoreMesh(core_axis_name="c", subcore_axis_name="s")

    @functools.partial(
        pl.kernel,
        mesh=mesh,
        out_type=[
            jax.ShapeDtypeStruct((BATCH, EMBED), jnp.float32),
            jax.ShapeDtypeStruct((BATCH, EMBED), jnp.float32),
        ],
        scratch_types=[
            pltpu.VMEM((_B_PER_W,), jnp.int32),
            pltpu.VMEM((_B_PER_W, EMBED), jnp.float32),
            pltpu.SemaphoreType.DMA,
        ],
    )
    def gather_k(users_hbm, movies_hbm, ut_hbm, mt_hbm, u_out, m_out,
                 idx_v, rows_v, sem):
        wid = lax.axis_index("s") * _NC + lax.axis_index("c")
        base = wid * _B_PER_W
        pltpu.sync_copy(users_hbm.at[pl.ds(base, _B_PER_W)], idx_v)
        pltpu.async_copy(ut_hbm.at[idx_v], rows_v, sem).wait()
        pltpu.sync_copy(rows_v, u_out.at[pl.ds(base, _B_PER_W)])
        pltpu.sync_copy(movies_hbm.at[pl.ds(base, _B_PER_W)], idx_v)
        pltpu.async_copy(mt_hbm.at[idx_v], rows_v, sem).wait()
        pltpu.sync_copy(rows_v, m_out.at[pl.ds(base, _B_PER_W)])

    return gather_k


_gather = _make_gather()

_BLK = 2048


def _mlp_body(u_ref, m_ref, w1a_ref, w1b_ref, b1_ref, w2_ref, b2_ref,
              w3t_ref, b3_ref, o_ref):
    h = jnp.dot(u_ref[...], w1a_ref[...], preferred_element_type=jnp.float32)
    h += jnp.dot(m_ref[...], w1b_ref[...], preferred_element_type=jnp.float32)
    h = jnp.maximum(h + b1_ref[...], 0.0)
    h = jnp.maximum(
        jnp.dot(h, w2_ref[...], preferred_element_type=jnp.float32)
        + b2_ref[...], 0.0)
    o_ref[...] = lax.dot_general(
        w3t_ref[...], h, (((1,), (1,)), ((), ())),
        preferred_element_type=jnp.float32) + b3_ref[0, 0]


def _mlp(u, m, W1, b1, W2, b2, W3, b3):
    w1a, w1b = W1[:EMBED], W1[EMBED:]
    grid = BATCH // _BLK
    out_row = pl.pallas_call(
        _mlp_body,
        grid=(grid,),
        in_specs=[
            pl.BlockSpec((_BLK, EMBED), lambda i: (i, 0)),
            pl.BlockSpec((_BLK, EMBED), lambda i: (i, 0)),
            pl.BlockSpec((EMBED, 128), lambda i: (0, 0)),
            pl.BlockSpec((EMBED, 128), lambda i: (0, 0)),
            pl.BlockSpec((1, 128), lambda i: (0, 0)),
            pl.BlockSpec((128, 64), lambda i: (0, 0)),
            pl.BlockSpec((1, 64), lambda i: (0, 0)),
            pl.BlockSpec((1, 64), lambda i: (0, 0)),
            pl.BlockSpec((1, 1), lambda i: (0, 0)),
        ],
        out_specs=pl.BlockSpec((1, _BLK), lambda i: (0, i)),
        out_shape=jax.ShapeDtypeStruct((1, BATCH), jnp.float32),
    )(u, m, w1a, w1b, b1.reshape(1, 128), W2, b2.reshape(1, 64),
      W3.reshape(1, 64), b3.reshape(1, 1))
    return out_row.reshape(BATCH, 1)


def kernel(users, movies, user_table, movie_table, W1, b1, W2, b2, W3, b3):
    u, m = _gather(users.astype(jnp.int32), movies.astype(jnp.int32),
                   user_table, movie_table)
    return _mlp(u, m, W1, b1, W2, b2, W3, b3)


# speedup vs baseline: 1.1215x; 1.0341x over previous
import functools

import jax
import jax.numpy as jnp
from jax import lax
from jax.experimental import pallas as pl
from jax.experimental.pallas import tpu as pltpu
from jax.experimental.pallas import tpu_sc as plsc

BATCH = 16384
EMBED = 128

_NC, _NS = 2, 16
_NW = _NC * _NS
_B_PER_W = BATCH // _NW


def _make_gather():
    mesh = plsc.VectorSubcoreMesh(core_axis_name="c", subcore_axis_name="s")

    @functools.partial(
        pl.kernel,
        mesh=mesh,
        out_type=[
            jax.ShapeDtypeStruct((BATCH, EMBED), jnp.float32),
            jax.ShapeDtypeStruct((BATCH, EMBED), jnp.float32),
        ],
        scratch_types=[
            pltpu.VMEM((_B_PER_W,), jnp.int32),
            pltpu.VMEM((_B_PER_W, EMBED), jnp.float32),
            pltpu.SemaphoreType.DMA,
        ],
    )
    def gather_k(users_hbm, movies_hbm, ut_hbm, mt_hbm, u_out, m_out,
                 idx_v, rows_v, sem):
        wid = lax.axis_index("s") * _NC + lax.axis_index("c")
        base = wid * _B_PER_W
        pltpu.sync_copy(users_hbm.at[pl.ds(base, _B_PER_W)], idx_v)
        pltpu.async_copy(ut_hbm.at[idx_v], rows_v, sem).wait()
        pltpu.sync_copy(rows_v, u_out.at[pl.ds(base, _B_PER_W)])
        pltpu.sync_copy(movies_hbm.at[pl.ds(base, _B_PER_W)], idx_v)
        pltpu.async_copy(mt_hbm.at[idx_v], rows_v, sem).wait()
        pltpu.sync_copy(rows_v, m_out.at[pl.ds(base, _B_PER_W)])

    return gather_k


_gather = _make_gather()

_BLK = 4096


def _mlp_body(u_ref, m_ref, w1a_ref, w1b_ref, b1_ref, w2_ref, b2_ref,
              w3t_ref, b3_ref, o_ref):
    h = jnp.dot(u_ref[...], w1a_ref[...], preferred_element_type=jnp.float32)
    h += jnp.dot(m_ref[...], w1b_ref[...], preferred_element_type=jnp.float32)
    h = jnp.maximum(h + b1_ref[...], 0.0)
    h = jnp.maximum(
        jnp.dot(h, w2_ref[...], preferred_element_type=jnp.float32)
        + b2_ref[...], 0.0)
    o_ref[...] = lax.dot_general(
        w3t_ref[...], h, (((1,), (1,)), ((), ())),
        preferred_element_type=jnp.float32) + b3_ref[0, 0]


def _mlp(u, m, W1, b1, W2, b2, W3, b3):
    w1a, w1b = W1[:EMBED], W1[EMBED:]
    grid = BATCH // _BLK
    out_row = pl.pallas_call(
        _mlp_body,
        grid=(grid,),
        in_specs=[
            pl.BlockSpec((_BLK, EMBED), lambda i: (i, 0)),
            pl.BlockSpec((_BLK, EMBED), lambda i: (i, 0)),
            pl.BlockSpec((EMBED, 128), lambda i: (0, 0)),
            pl.BlockSpec((EMBED, 128), lambda i: (0, 0)),
            pl.BlockSpec((1, 128), lambda i: (0, 0)),
            pl.BlockSpec((128, 64), lambda i: (0, 0)),
            pl.BlockSpec((1, 64), lambda i: (0, 0)),
            pl.BlockSpec((1, 64), lambda i: (0, 0)),
            pl.BlockSpec((1, 1), lambda i: (0, 0)),
        ],
        out_specs=pl.BlockSpec((1, _BLK), lambda i: (0, i)),
        out_shape=jax.ShapeDtypeStruct((1, BATCH), jnp.float32),
    )(u, m, w1a, w1b, b1.reshape(1, 128), W2, b2.reshape(1, 64),
      W3.reshape(1, 64), b3.reshape(1, 1))
    return out_row.reshape(BATCH, 1)


def kernel(users, movies, user_table, movie_table, W1, b1, W2, b2, W3, b3):
    u, m = _gather(users.astype(jnp.int32), movies.astype(jnp.int32),
                   user_table, movie_table)
    return _mlp(u, m, W1, b1, W2, b2, W3, b3)
